# baseline (speedup 1.0000x reference)
import jax
import jax.numpy as jnp
from jax.experimental import pallas as pl
from jax.experimental.pallas import tpu as pltpu

B = 16
N = 32768
D = 128
H = 256
ROWS_P = 8192
ROWS_M = 4096
NBLK_P = N // ROWS_P
NBLK_M = N // ROWS_M


def _pool_body(pf_ref, pfs_ref, x_ref, w1_ref, b1_ref, dg_ref, sacc_ref):
    j = pl.program_id(0)
    rows = j * ROWS_P + jax.lax.broadcasted_iota(jnp.int32, (ROWS_P, 1), 0)
    lo = (rows >= pfs_ref[...]).astype(jnp.float32)
    hi = (rows >= pf_ref[...]).astype(jnp.float32)
    part = jax.lax.dot_general(
        lo - hi, x_ref[...], (((0,), (0,)), ((), ())),
        preferred_element_type=jnp.float32,
                precision=jax.lax.Precision.HIGHEST)

    @pl.when(j == 0)
    def _():
        sacc_ref[...] = part

    @pl.when(j != 0)
    def _():
        sacc_ref[...] = sacc_ref[...] + part

    @pl.when(j == NBLK_P - 1)
    def _():
        g = jnp.dot(sacc_ref[...], w1_ref[D:, :],
                    preferred_element_type=jnp.float32,
                precision=jax.lax.Precision.HIGHEST) + b1_ref[...]
        dg_ref[...] = g - jnp.concatenate(
            [jnp.zeros((1, H), jnp.float32), g[: B - 1]], 0)


def _mlp_body(pfs_ref, x_ref, w1_ref, dg_ref, w2_ref, b2_ref, out_ref):
    j = pl.program_id(0)
    rows = j * ROWS_M + jax.lax.broadcasted_iota(jnp.int32, (ROWS_M, 1), 0)
    lo = (rows >= pfs_ref[...]).astype(jnp.float32)
    bias = jnp.dot(lo, dg_ref[...],
                   preferred_element_type=jnp.float32,
                precision=jax.lax.Precision.HIGHEST)
    h = jnp.maximum(
        jnp.dot(x_ref[...], w1_ref[:D, :],
                preferred_element_type=jnp.float32,
                precision=jax.lax.Precision.HIGHEST) + bias, 0.0)
    out_ref[...] = jnp.dot(h, w2_ref[...],
                           preferred_element_type=jnp.float32,
                precision=jax.lax.Precision.HIGHEST) + b2_ref[...]


@jax.jit
def _run(embed, prefix_sum, W1, b1, W2, b2):
    pfs = jnp.concatenate(
        [jnp.zeros((1, 1), prefix_sum.dtype),
         prefix_sum.reshape(1, B)[:, : B - 1]], axis=1)
    dg = pl.pallas_call(
        _pool_body,
        grid=(NBLK_P,),
        in_specs=[
            pl.BlockSpec((1, B), lambda j: (0, 0)),
            pl.BlockSpec((1, B), lambda j: (0, 0)),
            pl.BlockSpec((ROWS_P, D), lambda j: (j, 0)),
            pl.BlockSpec((2 * D, H), lambda j: (0, 0)),
            pl.BlockSpec((1, H), lambda j: (0, 0)),
        ],
        out_specs=pl.BlockSpec((B, H), lambda j: (0, 0)),
        out_shape=jax.ShapeDtypeStruct((B, H), jnp.float32),
        scratch_shapes=[pltpu.VMEM((B, D), jnp.float32)],
        compiler_params=pltpu.CompilerParams(
            dimension_semantics=("arbitrary",)),
    )(prefix_sum.reshape(1, B), pfs, embed, W1, b1.reshape(1, H))
    out = pl.pallas_call(
        _mlp_body,
        grid=(NBLK_M,),
        in_specs=[
            pl.BlockSpec((1, B), lambda j: (0, 0)),
            pl.BlockSpec((ROWS_M, D), lambda j: (j, 0)),
            pl.BlockSpec((2 * D, H), lambda j: (0, 0)),
            pl.BlockSpec((B, H), lambda j: (0, 0)),
            pl.BlockSpec((H, 1), lambda j: (0, 0)),
            pl.BlockSpec((1, 1), lambda j: (0, 0)),
        ],
        out_specs=pl.BlockSpec((ROWS_M, 1), lambda j: (j, 0)),
        out_shape=jax.ShapeDtypeStruct((N, 1), jnp.float32),
        compiler_params=pltpu.CompilerParams(
            dimension_semantics=("arbitrary",)),
    )(pfs, embed, W1, dg, W2, b2.reshape(1, 1))
    return out


def kernel(embed, prefix_sum, W1, b1, W2, b2):
    return (_run(embed, prefix_sum, W1, b1, W2, b2), prefix_sum)


# speedup vs baseline: 2.2701x; 2.2701x over previous
import jax
import jax.numpy as jnp
from jax.experimental import pallas as pl
from jax.experimental.pallas import tpu as pltpu

B = 16
N = 32768
D = 128
H = 256
ROWS_P = 8192
ROWS_M = 4096
NBLK_P = N // ROWS_P
NBLK_M = N // ROWS_M


def _bdot(a, b):
    return jax.lax.dot_general(
        a.astype(jnp.bfloat16), b.astype(jnp.bfloat16),
        (((a.ndim - 1,), (0,)), ((), ())),
        preferred_element_type=jnp.float32)


def _pool_body(pf_ref, pfs_ref, x_ref, w1_ref, b1_ref, dg_ref, sacc_ref):
    j = pl.program_id(0)
    rows = j * ROWS_P + jax.lax.broadcasted_iota(jnp.int32, (ROWS_P, 1), 0)
    lo = (rows >= pfs_ref[...]).astype(jnp.float32)
    hi = (rows >= pf_ref[...]).astype(jnp.float32)
    part = jax.lax.dot_general(
        lo - hi, x_ref[...], (((0,), (0,)), ((), ())),
        preferred_element_type=jnp.float32,
        precision=jax.lax.Precision.HIGHEST)

    @pl.when(j == 0)
    def _():
        sacc_ref[...] = part

    @pl.when(j != 0)
    def _():
        sacc_ref[...] = sacc_ref[...] + part

    @pl.when(j == NBLK_P - 1)
    def _():
        g = _bdot(sacc_ref[...], w1_ref[D:, :]) + b1_ref[...]
        dg_ref[...] = g - jnp.concatenate(
            [jnp.zeros((1, H), jnp.float32), g[: B - 1]], 0)


def _mlp_body(pfs_ref, x_ref, w1_ref, dg_ref, w2_ref, b2_ref, out_ref):
    j = pl.program_id(0)
    rows = j * ROWS_M + jax.lax.broadcasted_iota(jnp.int32, (ROWS_M, 1), 0)
    lo = (rows >= pfs_ref[...]).astype(jnp.float32)
    bias = jnp.dot(lo, dg_ref[...],
                   preferred_element_type=jnp.float32,
                   precision=jax.lax.Precision.HIGHEST)
    h = jnp.maximum(_bdot(x_ref[...], w1_ref[:D, :]) + bias, 0.0)
    out_ref[...] = _bdot(h, w2_ref[...]) + b2_ref[...]


@jax.jit
def _run(embed, prefix_sum, W1, b1, W2, b2):
    pfs = jnp.concatenate(
        [jnp.zeros((1, 1), prefix_sum.dtype),
         prefix_sum.reshape(1, B)[:, : B - 1]], axis=1)
    dg = pl.pallas_call(
        _pool_body,
        grid=(NBLK_P,),
        in_specs=[
            pl.BlockSpec((1, B), lambda j: (0, 0)),
            pl.BlockSpec((1, B), lambda j: (0, 0)),
            pl.BlockSpec((ROWS_P, D), lambda j: (j, 0)),
            pl.BlockSpec((2 * D, H), lambda j: (0, 0)),
            pl.BlockSpec((1, H), lambda j: (0, 0)),
        ],
        out_specs=pl.BlockSpec((B, H), lambda j: (0, 0)),
        out_shape=jax.ShapeDtypeStruct((B, H), jnp.float32),
        scratch_shapes=[pltpu.VMEM((B, D), jnp.float32)],
        compiler_params=pltpu.CompilerParams(
            dimension_semantics=("arbitrary",)),
    )(prefix_sum.reshape(1, B), pfs, embed, W1, b1.reshape(1, H))
    out = pl.pallas_call(
        _mlp_body,
        grid=(NBLK_M,),
        in_specs=[
            pl.BlockSpec((1, B), lambda j: (0, 0)),
            pl.BlockSpec((ROWS_M, D), lambda j: (j, 0)),
            pl.BlockSpec((2 * D, H), lambda j: (0, 0)),
            pl.BlockSpec((B, H), lambda j: (0, 0)),
            pl.BlockSpec((H, 1), lambda j: (0, 0)),
            pl.BlockSpec((1, 1), lambda j: (0, 0)),
        ],
        out_specs=pl.BlockSpec((ROWS_M, 1), lambda j: (j, 0)),
        out_shape=jax.ShapeDtypeStruct((N, 1), jnp.float32),
        compiler_params=pltpu.CompilerParams(
            dimension_semantics=("arbitrary",)),
    )(pfs, embed, W1, dg, W2, b2.reshape(1, 1))
    return out


def kernel(embed, prefix_sum, W1, b1, W2, b2):
    return (_run(embed, prefix_sum, W1, b1, W2, b2), prefix_sum)


# speedup vs baseline: 3.2997x; 1.4535x over previous
import jax
import jax.numpy as jnp
from jax.experimental import pallas as pl
from jax.experimental.pallas import tpu as pltpu

B = 16
N = 32768
D = 128
H = 256
ROWS_P = 8192
ROWS_M = 4096
NBLK_P = N // ROWS_P
NBLK_M = N // ROWS_M


def _bdot(a, b):
    return jax.lax.dot_general(
        a.astype(jnp.bfloat16), b.astype(jnp.bfloat16),
        (((a.ndim - 1,), (0,)), ((), ())),
        preferred_element_type=jnp.float32)


def _pool_body(pf_ref, pfs_ref, x_ref, w1_ref, b1_ref, dg_ref, sacc_ref):
    j = pl.program_id(0)
    rows = j * ROWS_P + jax.lax.broadcasted_iota(jnp.int32, (ROWS_P, 1), 0)
    lo = (rows >= pfs_ref[...]).astype(jnp.float32)
    hi = (rows >= pf_ref[...]).astype(jnp.float32)
    oh = (lo - hi).astype(jnp.bfloat16)
    x = x_ref[...]
    xh = x.astype(jnp.bfloat16)
    r = x - xh.astype(jnp.float32)
    xm = r.astype(jnp.bfloat16)
    xl = (r - xm.astype(jnp.float32)).astype(jnp.bfloat16)
    dims = (((0,), (0,)), ((), ()))
    part = (jax.lax.dot_general(oh, xh, dims,
                                preferred_element_type=jnp.float32)
            + jax.lax.dot_general(oh, xm, dims,
                                  preferred_element_type=jnp.float32)
            + jax.lax.dot_general(oh, xl, dims,
                                  preferred_element_type=jnp.float32))

    @pl.when(j == 0)
    def _():
        sacc_ref[...] = part

    @pl.when(j != 0)
    def _():
        sacc_ref[...] = sacc_ref[...] + part

    @pl.when(j == NBLK_P - 1)
    def _():
        g = _bdot(sacc_ref[...], w1_ref[D:, :]) + b1_ref[...]
        dg = g - jnp.concatenate(
            [jnp.zeros((1, H), jnp.float32), g[: B - 1]], 0)
        d0 = dg.astype(jnp.bfloat16).astype(jnp.float32)
        r1 = dg - d0
        d1 = r1.astype(jnp.bfloat16).astype(jnp.float32)
        d2 = (r1 - d1).astype(jnp.bfloat16).astype(jnp.float32)
        dg_ref[...] = jnp.concatenate([d0, d1, d2], 0)


def _mlp_body(pfs_ref, x_ref, w1_ref, dg_ref, w2_ref, b2_ref, out_ref):
    j = pl.program_id(0)
    rows = j * ROWS_M + jax.lax.broadcasted_iota(jnp.int32, (ROWS_M, 1), 0)
    lo3 = (rows >= pfs_ref[...]).astype(jnp.float32)
    bias = jnp.dot(lo3, dg_ref[...],
                   preferred_element_type=jnp.float32)
    h = jnp.maximum(_bdot(x_ref[...], w1_ref[:D, :]) + bias, 0.0)
    out_ref[...] = _bdot(h, w2_ref[...]) + b2_ref[...]


@jax.jit
def _run(embed, prefix_sum, W1, b1, W2, b2):
    pfs = jnp.concatenate(
        [jnp.zeros((1, 1), prefix_sum.dtype),
         prefix_sum.reshape(1, B)[:, : B - 1]], axis=1)
    dg = pl.pallas_call(
        _pool_body,
        grid=(NBLK_P,),
        in_specs=[
            pl.BlockSpec((1, B), lambda j: (0, 0)),
            pl.BlockSpec((1, B), lambda j: (0, 0)),
            pl.BlockSpec((ROWS_P, D), lambda j: (j, 0)),
            pl.BlockSpec((2 * D, H), lambda j: (0, 0)),
            pl.BlockSpec((1, H), lambda j: (0, 0)),
        ],
        out_specs=pl.BlockSpec((3 * B, H), lambda j: (0, 0)),
        out_shape=jax.ShapeDtypeStruct((3 * B, H), jnp.float32),
        scratch_shapes=[pltpu.VMEM((B, D), jnp.float32)],
        compiler_params=pltpu.CompilerParams(
            dimension_semantics=("arbitrary",)),
    )(prefix_sum.reshape(1, B), pfs, embed, W1, b1.reshape(1, H))
    out = pl.pallas_call(
        _mlp_body,
        grid=(NBLK_M,),
        in_specs=[
            pl.BlockSpec((1, 3 * B), lambda j: (0, 0)),
            pl.BlockSpec((ROWS_M, D), lambda j: (j, 0)),
            pl.BlockSpec((2 * D, H), lambda j: (0, 0)),
            pl.BlockSpec((3 * B, H), lambda j: (0, 0)),
            pl.BlockSpec((H, 1), lambda j: (0, 0)),
            pl.BlockSpec((1, 1), lambda j: (0, 0)),
        ],
        out_specs=pl.BlockSpec((ROWS_M, 1), lambda j: (j, 0)),
        out_shape=jax.ShapeDtypeStruct((N, 1), jnp.float32),
        compiler_params=pltpu.CompilerParams(
            dimension_semantics=("arbitrary",)),
    )(jnp.tile(pfs, (1, 3)), embed, W1, dg, W2, b2.reshape(1, 1))
    return out


def kernel(embed, prefix_sum, W1, b1, W2, b2):
    return (_run(embed, prefix_sum, W1, b1, W2, b2), prefix_sum)


# speedup vs baseline: 3.4829x; 1.0555x over previous
import jax
import jax.numpy as jnp
from jax.experimental import pallas as pl
from jax.experimental.pallas import tpu as pltpu

B = 16
N = 32768
D = 128
H = 256
ROWS = 4096
NBLK = N // ROWS


def _bdot(a, b):
    return jax.lax.dot_general(
        a.astype(jnp.bfloat16), b.astype(jnp.bfloat16),
        (((a.ndim - 1,), (0,)), ((), ())),
        preferred_element_type=jnp.float32)


def _body(pf_ref, pfs3_ref, x_ref, w1_ref, b1_ref, w2_ref, b2_ref, out_ref,
          stash_ref, sacc_ref, dgt_ref):
    j = pl.program_id(0)

    @pl.when(j < NBLK)
    def _pool():
        rows = j * ROWS + jax.lax.broadcasted_iota(jnp.int32, (ROWS, 1), 0)
        lo = (rows >= pfs3_ref[0:1, :B]).astype(jnp.float32)
        hi = (rows >= pf_ref[...]).astype(jnp.float32)
        oh = (lo - hi).astype(jnp.bfloat16)
        x = x_ref[...]
        xh = x.astype(jnp.bfloat16)
        stash_ref[pl.ds(j * ROWS, ROWS), :] = xh
        r = x - xh.astype(jnp.float32)
        xm = r.astype(jnp.bfloat16)
        xl = (r - xm.astype(jnp.float32)).astype(jnp.bfloat16)
        dims = (((0,), (0,)), ((), ()))
        part = (jax.lax.dot_general(oh, xh, dims,
                                    preferred_element_type=jnp.float32)
                + jax.lax.dot_general(oh, xm, dims,
                                      preferred_element_type=jnp.float32)
                + jax.lax.dot_general(oh, xl, dims,
                                      preferred_element_type=jnp.float32))

        @pl.when(j == 0)
        def _():
            sacc_ref[...] = part

        @pl.when(j != 0)
        def _():
            sacc_ref[...] = sacc_ref[...] + part

        @pl.when(j == NBLK - 1)
        def _():
            g = _bdot(sacc_ref[...], w1_ref[D:, :]) + b1_ref[...]
            dg = g - jnp.concatenate(
                [jnp.zeros((1, H), jnp.float32), g[: B - 1]], 0)
            d0 = dg.astype(jnp.bfloat16).astype(jnp.float32)
            r1 = dg - d0
            d1 = r1.astype(jnp.bfloat16).astype(jnp.float32)
            d2 = (r1 - d1).astype(jnp.bfloat16).astype(jnp.float32)
            dgt_ref[...] = jnp.concatenate([d0, d1, d2], 0)

    @pl.when(j >= NBLK)
    def _mlp():
        k = j - NBLK
        rows = k * ROWS + jax.lax.broadcasted_iota(jnp.int32, (ROWS, 1), 0)
        lo3 = (rows >= pfs3_ref[...]).astype(jnp.bfloat16)
        bias = jax.lax.dot_general(
            lo3, dgt_ref[...].astype(jnp.bfloat16),
            (((1,), (0,)), ((), ())),
            preferred_element_type=jnp.float32)
        xb = stash_ref[pl.ds(k * ROWS, ROWS), :]
        h = jnp.maximum(
            jax.lax.dot_general(xb, w1_ref[:D, :].astype(jnp.bfloat16),
                                (((1,), (0,)), ((), ())),
                                preferred_element_type=jnp.float32) + bias,
            0.0)
        out_ref[...] = _bdot(h, w2_ref[...]) + b2_ref[...]


@jax.jit
def _run(embed, prefix_sum, W1, b1, W2, b2):
    pf = prefix_sum.reshape(1, B)
    pfs = jnp.concatenate(
        [jnp.zeros((1, 1), prefix_sum.dtype), pf[:, : B - 1]], axis=1)
    pfs3 = jnp.tile(pfs, (1, 3))
    out = pl.pallas_call(
        _body,
        grid=(2 * NBLK,),
        in_specs=[
            pl.BlockSpec((1, B), lambda j: (0, 0)),
            pl.BlockSpec((1, 3 * B), lambda j: (0, 0)),
            pl.BlockSpec((ROWS, D), lambda j: (jnp.minimum(j, NBLK - 1), 0)),
            pl.BlockSpec((2 * D, H), lambda j: (0, 0)),
            pl.BlockSpec((1, H), lambda j: (0, 0)),
            pl.BlockSpec((H, 1), lambda j: (0, 0)),
            pl.BlockSpec((1, 1), lambda j: (0, 0)),
        ],
        out_specs=pl.BlockSpec(
            (ROWS, 1), lambda j: (jnp.maximum(j - NBLK, 0), 0)),
        out_shape=jax.ShapeDtypeStruct((N, 1), jnp.float32),
        scratch_shapes=[
            pltpu.VMEM((N, D), jnp.bfloat16),
            pltpu.VMEM((B, D), jnp.float32),
            pltpu.VMEM((3 * B, H), jnp.float32),
        ],
        compiler_params=pltpu.CompilerParams(
            dimension_semantics=("arbitrary",)),
    )(pf, pfs3, embed, W1, b1.reshape(1, H), W2, b2.reshape(1, 1))
    return out


def kernel(embed, prefix_sum, W1, b1, W2, b2):
    return (_run(embed, prefix_sum, W1, b1, W2, b2), prefix_sum)


# speedup vs baseline: 3.7727x; 1.0832x over previous
import jax
import jax.numpy as jnp
from jax.experimental import pallas as pl
from jax.experimental.pallas import tpu as pltpu

B = 16
N = 32768
D = 128
H = 256
ROWS_A = 4096
ROWS_B = 8192
NBLK_A = N // ROWS_A
NBLK_B = N // ROWS_B


def _bdot(a, b):
    return jax.lax.dot_general(
        a.astype(jnp.bfloat16), b.astype(jnp.bfloat16),
        (((a.ndim - 1,), (0,)), ((), ())),
        preferred_element_type=jnp.float32)


def _body(pf_ref, pfs3_ref, x_ref, w1_ref, b1_ref, w2_ref, b2_ref, out_ref,
          px_ref, sacc_ref, dgt_ref):
    j = pl.program_id(0)

    @pl.when(j < NBLK_A)
    def _phase_a():
        rows = j * ROWS_A + jax.lax.broadcasted_iota(jnp.int32, (ROWS_A, 1), 0)
        lo = (rows >= pfs3_ref[0:1, :B]).astype(jnp.float32)
        hi = (rows >= pf_ref[...]).astype(jnp.float32)
        oh = (lo - hi).astype(jnp.bfloat16)
        x = x_ref[...]
        xh = x.astype(jnp.bfloat16)
        px_ref[pl.ds(j * ROWS_A, ROWS_A), :] = jax.lax.dot_general(
            xh, w1_ref[:D, :].astype(jnp.bfloat16),
            (((1,), (0,)), ((), ())), preferred_element_type=jnp.float32)
        r = x - xh.astype(jnp.float32)
        xm = r.astype(jnp.bfloat16)
        xl = (r - xm.astype(jnp.float32)).astype(jnp.bfloat16)
        dims = (((0,), (0,)), ((), ()))
        part = (jax.lax.dot_general(oh, xh, dims,
                                    preferred_element_type=jnp.float32)
                + jax.lax.dot_general(oh, xm, dims,
                                      preferred_element_type=jnp.float32)
                + jax.lax.dot_general(oh, xl, dims,
                                      preferred_element_type=jnp.float32))

        @pl.when(j == 0)
        def _():
            sacc_ref[...] = part

        @pl.when(j != 0)
        def _():
            sacc_ref[...] = sacc_ref[...] + part

        @pl.when(j == NBLK_A - 1)
        def _():
            g = _bdot(sacc_ref[...], w1_ref[D:, :]) + b1_ref[...]
            dg = g - jnp.concatenate(
                [jnp.zeros((1, H), jnp.float32), g[: B - 1]], 0)
            d0 = dg.astype(jnp.bfloat16).astype(jnp.float32)
            r1 = dg - d0
            d1 = r1.astype(jnp.bfloat16).astype(jnp.float32)
            d2 = (r1 - d1).astype(jnp.bfloat16).astype(jnp.float32)
            dgt_ref[...] = jnp.concatenate([d0, d1, d2], 0)

    @pl.when(j >= NBLK_A)
    def _phase_b():
        k = j - NBLK_A
        rows = k * ROWS_B + jax.lax.broadcasted_iota(jnp.int32, (ROWS_B, 1), 0)
        lo3 = (rows >= pfs3_ref[...]).astype(jnp.bfloat16)
        bias = jax.lax.dot_general(
            lo3, dgt_ref[...].astype(jnp.bfloat16),
            (((1,), (0,)), ((), ())),
            preferred_element_type=jnp.float32)
        h = jnp.maximum(px_ref[pl.ds(k * ROWS_B, ROWS_B), :] + bias, 0.0)
        out_ref[...] = _bdot(h, w2_ref[...]) + b2_ref[...]


@jax.jit
def _run(embed, prefix_sum, W1, b1, W2, b2):
    pf = prefix_sum.reshape(1, B)
    pfs = jnp.concatenate(
        [jnp.zeros((1, 1), prefix_sum.dtype), pf[:, : B - 1]], axis=1)
    pfs3 = jnp.tile(pfs, (1, 3))
    out = pl.pallas_call(
        _body,
        grid=(NBLK_A + NBLK_B,),
        in_specs=[
            pl.BlockSpec((1, B), lambda j: (0, 0)),
            pl.BlockSpec((1, 3 * B), lambda j: (0, 0)),
            pl.BlockSpec((ROWS_A, D),
                         lambda j: (jnp.minimum(j, NBLK_A - 1), 0)),
            pl.BlockSpec((2 * D, H), lambda j: (0, 0)),
            pl.BlockSpec((1, H), lambda j: (0, 0)),
            pl.BlockSpec((H, 1), lambda j: (0, 0)),
            pl.BlockSpec((1, 1), lambda j: (0, 0)),
        ],
        out_specs=pl.BlockSpec(
            (ROWS_B, 1), lambda j: (jnp.maximum(j - NBLK_A, 0), 0)),
        out_shape=jax.ShapeDtypeStruct((N, 1), jnp.float32),
        scratch_shapes=[
            pltpu.VMEM((N, H), jnp.float32),
            pltpu.VMEM((B, D), jnp.float32),
            pltpu.VMEM((3 * B, H), jnp.float32),
        ],
        compiler_params=pltpu.CompilerParams(
            dimension_semantics=("arbitrary",)),
    )(pf, pfs3, embed, W1, b1.reshape(1, H), W2, b2.reshape(1, 1))
    return out


def kernel(embed, prefix_sum, W1, b1, W2, b2):
    return (_run(embed, prefix_sum, W1, b1, W2, b2), prefix_sum)


# speedup vs baseline: 3.8766x; 1.0275x over previous
import jax
import jax.numpy as jnp
from jax.experimental import pallas as pl
from jax.experimental.pallas import tpu as pltpu

B = 16
N = 32768
D = 128
H = 256
ROWS_A = 8192
ROWS_B = 8192
NBLK_A = N // ROWS_A
NBLK_B = N // ROWS_B


def _bdot(a, b):
    return jax.lax.dot_general(
        a.astype(jnp.bfloat16), b.astype(jnp.bfloat16),
        (((a.ndim - 1,), (0,)), ((), ())),
        preferred_element_type=jnp.float32)


def _body(pf_ref, pfs3_ref, x_ref, w1_ref, b1_ref, w2_ref, b2_ref, out_ref,
          px_ref, sacc_ref, dgt_ref):
    j = pl.program_id(0)

    @pl.when(j < NBLK_A)
    def _phase_a():
        rows = j * ROWS_A + jax.lax.broadcasted_iota(jnp.int32, (ROWS_A, 1), 0)
        lo = (rows >= pfs3_ref[0:1, :B]).astype(jnp.float32)
        hi = (rows >= pf_ref[...]).astype(jnp.float32)
        oh = (lo - hi).astype(jnp.bfloat16)
        x = x_ref[...]
        xh = x.astype(jnp.bfloat16)
        px_ref[pl.ds(j * ROWS_A, ROWS_A), :] = jax.lax.dot_general(
            xh, w1_ref[:D, :].astype(jnp.bfloat16),
            (((1,), (0,)), ((), ())),
            preferred_element_type=jnp.float32).astype(jnp.bfloat16)
        xm = (x - xh.astype(jnp.float32)).astype(jnp.bfloat16)
        dims = (((0,), (0,)), ((), ()))
        part = (jax.lax.dot_general(oh, xh, dims,
                                    preferred_element_type=jnp.float32)
                + jax.lax.dot_general(oh, xm, dims,
                                      preferred_element_type=jnp.float32))

        @pl.when(j == 0)
        def _():
            sacc_ref[...] = part

        @pl.when(j != 0)
        def _():
            sacc_ref[...] = sacc_ref[...] + part

        @pl.when(j == NBLK_A - 1)
        def _():
            g = _bdot(sacc_ref[...], w1_ref[D:, :]) + b1_ref[...]
            dg = g - jnp.concatenate(
                [jnp.zeros((1, H), jnp.float32), g[: B - 1]], 0)
            d0 = dg.astype(jnp.bfloat16).astype(jnp.float32)
            r1 = dg - d0
            d1 = r1.astype(jnp.bfloat16).astype(jnp.float32)
            d2 = (r1 - d1).astype(jnp.bfloat16).astype(jnp.float32)
            dgt_ref[...] = jnp.concatenate([d0, d1, d2], 0)

    @pl.when(j >= NBLK_A)
    def _phase_b():
        k = j - NBLK_A
        rows = k * ROWS_B + jax.lax.broadcasted_iota(jnp.int32, (ROWS_B, 1), 0)
        lo3 = (rows >= pfs3_ref[...]).astype(jnp.bfloat16)
        bias = jax.lax.dot_general(
            lo3, dgt_ref[...].astype(jnp.bfloat16),
            (((1,), (0,)), ((), ())),
            preferred_element_type=jnp.float32)
        h = jnp.maximum(px_ref[pl.ds(k * ROWS_B, ROWS_B), :] + bias, 0.0)
        out_ref[...] = _bdot(h, w2_ref[...]) + b2_ref[...]


@jax.jit
def _run(embed, prefix_sum, W1, b1, W2, b2):
    pf = prefix_sum.reshape(1, B)
    pfs = jnp.concatenate(
        [jnp.zeros((1, 1), prefix_sum.dtype), pf[:, : B - 1]], axis=1)
    pfs3 = jnp.tile(pfs, (1, 3))
    out = pl.pallas_call(
        _body,
        grid=(NBLK_A + NBLK_B,),
        in_specs=[
            pl.BlockSpec((1, B), lambda j: (0, 0)),
            pl.BlockSpec((1, 3 * B), lambda j: (0, 0)),
            pl.BlockSpec((ROWS_A, D),
                         lambda j: (jnp.minimum(j, NBLK_A - 1), 0)),
            pl.BlockSpec((2 * D, H), lambda j: (0, 0)),
            pl.BlockSpec((1, H), lambda j: (0, 0)),
            pl.BlockSpec((H, 1), lambda j: (0, 0)),
            pl.BlockSpec((1, 1), lambda j: (0, 0)),
        ],
        out_specs=pl.BlockSpec(
            (ROWS_B, 1), lambda j: (jnp.maximum(j - NBLK_A, 0), 0)),
        out_shape=jax.ShapeDtypeStruct((N, 1), jnp.float32),
        scratch_shapes=[
            pltpu.VMEM((N, H), jnp.bfloat16),
            pltpu.VMEM((B, D), jnp.float32),
            pltpu.VMEM((3 * B, H), jnp.float32),
        ],
        compiler_params=pltpu.CompilerParams(
            dimension_semantics=("arbitrary",)),
    )(pf, pfs3, embed, W1, b1.reshape(1, H), W2, b2.reshape(1, 1))
    return out


def kernel(embed, prefix_sum, W1, b1, W2, b2):
    return (_run(embed, prefix_sum, W1, b1, W2, b2), prefix_sum)
